# baseline (speedup 1.0000x reference)
import jax
import jax.numpy as jnp
from jax import lax
from jax.experimental import pallas as pl
from jax.experimental.pallas import tpu as pltpu
from jax.experimental.pallas import tpu_sc as plsc

_K = 5
_L = 16
_GATHER_WORKERS = 8


def _sc_gather_body(pred_hbm, gt_hbm, v_hbm, gt_v, win_v, out_v, sem):
    lanes = lax.iota(jnp.int32, _L)
    wid = lax.axis_index("s") * 2 + lax.axis_index("c")

    @pl.when(wid < _GATHER_WORKERS)
    def _():
        base = wid * _L
        pltpu.async_copy(gt_hbm.at[pl.ds(base, _L)], gt_v, sem).wait()
        gvec = gt_v[...]
        copies = []
        for j in range(_L):
            g = jnp.sum(jnp.where(lanes == j, gvec, 0))
            start = (g // _L) * _L
            copies.append(
                pltpu.make_async_copy(
                    pred_hbm.at[base + j, pl.ds(start, _L)],
                    win_v.at[pl.ds(j * _L, _L)],
                    sem,
                )
            )
        for c in copies:
            c.start()
        for c in copies:
            c.wait()
        acc = jnp.zeros((_L,), jnp.float32)
        for j in range(_L):
            g = jnp.sum(jnp.where(lanes == j, gvec, 0))
            win = win_v[pl.ds(j * _L, _L)]
            val = jnp.sum(jnp.where(lanes == g % _L, win, 0.0))
            acc = acc + jnp.where(lanes == j, val, 0.0)
        out_v[...] = acc
        pltpu.async_copy(out_v, v_hbm.at[pl.ds(base, _L)], sem).wait()


def _make_sc_gather(b):
    mesh = plsc.VectorSubcoreMesh(core_axis_name="c", subcore_axis_name="s")
    cp = pltpu.CompilerParams(use_tc_tiling_on_sc=True,
                              needs_layout_passes=False)
    return pl.kernel(
        _sc_gather_body,
        out_type=jax.ShapeDtypeStruct((b,), jnp.float32),
        mesh=mesh,
        scratch_types=[
            pltpu.VMEM((_L,), jnp.int32),
            pltpu.VMEM((_L * _L,), jnp.float32),
            pltpu.VMEM((_L,), jnp.float32),
            pltpu.SemaphoreType.DMA,
        ],
        compiler_params=cp,
    )


def _acc_body(gt_ref, v_ref, pred_ref, out_ref):
    i = pl.program_id(0)
    pred = pred_ref[...]
    g = gt_ref[...]
    v = v_ref[...]
    rb, n = pred.shape
    cnt_gt = jnp.sum((pred > v).astype(jnp.int32), axis=1)
    cnt_eq = jnp.sum((pred == v).astype(jnp.int32), axis=1)

    @pl.when(i == 0)
    def _():
        out_ref[...] = jnp.zeros((1, 1), jnp.float32)

    ambiguous = jnp.any((cnt_gt < _K) & (cnt_gt + cnt_eq - 1 >= _K))

    @pl.when(jnp.logical_not(ambiguous))
    def _():
        part = jnp.sum((cnt_gt < _K).astype(jnp.float32)).reshape(1, 1)
        out_ref[...] += part

    @pl.when(ambiguous)
    def _():
        col = jax.lax.broadcasted_iota(jnp.int32, (rb, n), 1)
        cnt_eq_low = jnp.sum(((pred == v) & (col < g)).astype(jnp.int32),
                             axis=1)
        part = jnp.sum(((cnt_gt + cnt_eq_low) < _K)
                       .astype(jnp.float32)).reshape(1, 1)
        out_ref[...] += part


def kernel(pred, gt):
    b, n = pred.shape
    v = _make_sc_gather(b)(pred, gt)
    rb = 16
    grid = (b // rb,)
    out = pl.pallas_call(
        _acc_body,
        grid=grid,
        in_specs=[
            pl.BlockSpec((rb, 1), lambda i: (i, 0)),
            pl.BlockSpec((rb, 1), lambda i: (i, 0)),
            pl.BlockSpec((rb, n), lambda i: (i, 0)),
        ],
        out_specs=pl.BlockSpec((1, 1), lambda i: (0, 0)),
        out_shape=jax.ShapeDtypeStruct((1, 1), jnp.float32),
    )(gt.reshape(b, 1), v.reshape(b, 1), pred)
    return out[0, 0] / b


# speedup vs baseline: 2.6287x; 2.6287x over previous
import jax
import jax.numpy as jnp
from jax.experimental import pallas as pl

_K = 5


def _acc_body(gt_ref, pred_ref, out_ref):
    i = pl.program_id(0)
    pred = pred_ref[...]
    g = gt_ref[...]
    rb, n = pred.shape
    col = jax.lax.broadcasted_iota(jnp.int32, (rb, n), 1)
    v = jnp.max(jnp.where(col == g, pred, -jnp.inf), axis=1, keepdims=True)
    cnt_gt = jnp.sum((pred > v).astype(jnp.int32), axis=1)
    cnt_eq = jnp.sum((pred == v).astype(jnp.int32), axis=1)

    @pl.when(i == 0)
    def _():
        out_ref[...] = jnp.zeros((1, 1), jnp.float32)

    ambiguous = jnp.any((cnt_gt < _K) & (cnt_gt + cnt_eq - 1 >= _K))

    @pl.when(jnp.logical_not(ambiguous))
    def _():
        part = jnp.sum((cnt_gt < _K).astype(jnp.float32)).reshape(1, 1)
        out_ref[...] += part

    @pl.when(ambiguous)
    def _():
        cnt_eq_low = jnp.sum(((pred == v) & (col < g)).astype(jnp.int32),
                             axis=1)
        part = jnp.sum(((cnt_gt + cnt_eq_low) < _K)
                       .astype(jnp.float32)).reshape(1, 1)
        out_ref[...] += part


def kernel(pred, gt):
    b, n = pred.shape
    rb = 32
    grid = (b // rb,)
    out = pl.pallas_call(
        _acc_body,
        grid=grid,
        in_specs=[
            pl.BlockSpec((rb, 1), lambda i: (i, 0)),
            pl.BlockSpec((rb, n), lambda i: (i, 0)),
        ],
        out_specs=pl.BlockSpec((1, 1), lambda i: (0, 0)),
        out_shape=jax.ShapeDtypeStruct((1, 1), jnp.float32),
    )(gt.reshape(b, 1), pred)
    return out[0, 0] / b


# speedup vs baseline: 2.6811x; 1.0199x over previous
import jax
import jax.numpy as jnp
from jax.experimental import pallas as pl

_K = 5


def _acc_body(gt_ref, pred_ref, out_ref):
    i = pl.program_id(0)
    pred = pred_ref[...]
    g = gt_ref[...]
    rb, n = pred.shape
    col = jax.lax.broadcasted_iota(jnp.int32, (rb, n), 1)
    v = jnp.max(jnp.where(col == g, pred, -jnp.inf), axis=1, keepdims=True)
    cnt_gt = jnp.sum((pred > v).astype(jnp.int32), axis=1)
    cnt_eq = jnp.sum((pred == v).astype(jnp.int32), axis=1)

    @pl.when(i == 0)
    def _():
        out_ref[...] = jnp.zeros((1, 1), jnp.float32)

    ambiguous = jnp.any((cnt_gt < _K) & (cnt_gt + cnt_eq - 1 >= _K))

    @pl.when(jnp.logical_not(ambiguous))
    def _():
        part = jnp.sum((cnt_gt < _K).astype(jnp.float32)).reshape(1, 1)
        out_ref[...] += part

    @pl.when(ambiguous)
    def _():
        cnt_eq_low = jnp.sum(((pred == v) & (col < g)).astype(jnp.int32),
                             axis=1)
        part = jnp.sum(((cnt_gt + cnt_eq_low) < _K)
                       .astype(jnp.float32)).reshape(1, 1)
        out_ref[...] += part


def kernel(pred, gt):
    b, n = pred.shape
    rb = 64
    grid = (b // rb,)
    out = pl.pallas_call(
        _acc_body,
        grid=grid,
        in_specs=[
            pl.BlockSpec((rb, 1), lambda i: (i, 0)),
            pl.BlockSpec((rb, n), lambda i: (i, 0)),
        ],
        out_specs=pl.BlockSpec((1, 1), lambda i: (0, 0)),
        out_shape=jax.ShapeDtypeStruct((1, 1), jnp.float32),
    )(gt.reshape(b, 1), pred)
    return out[0, 0] / b


# speedup vs baseline: 2.9001x; 1.0817x over previous
import jax
import jax.numpy as jnp
from jax import lax
from jax.experimental import pallas as pl
from jax.experimental.pallas import tpu as pltpu

_K = 5


def _acc_body(gt_sm_ref, gt_ref, pred_ref, out_ref):
    i = pl.program_id(0)
    pred = pred_ref[...]
    g = gt_ref[...]
    rb, n = pred.shape
    sub_iota = lax.broadcasted_iota(jnp.int32, (8, 128), 0)
    lane_iota = lax.broadcasted_iota(jnp.int32, (8, 128), 1)
    row_iota = lax.broadcasted_iota(jnp.int32, (rb, 1), 0)

    v = jnp.zeros((rb, 1), jnp.float32)
    for r in range(rb):
        gr = gt_sm_ref[r, 0]
        cb = pl.multiple_of((gr // 128) * 128, 128)
        tile = pred_ref[pl.ds((r // 8) * 8, 8), pl.ds(cb, 128)]
        val = jnp.sum(jnp.where((sub_iota == r % 8) & (lane_iota == gr % 128),
                                tile, 0.0))
        v = v + jnp.where(row_iota == r, val, 0.0)

    cnt_gt = jnp.sum((pred > v).astype(jnp.int32), axis=1)
    cnt_eq = jnp.sum((pred == v).astype(jnp.int32), axis=1)

    @pl.when(i == 0)
    def _():
        out_ref[...] = jnp.zeros((1, 1), jnp.float32)

    ambiguous = jnp.any((cnt_gt < _K) & (cnt_gt + cnt_eq - 1 >= _K))

    @pl.when(jnp.logical_not(ambiguous))
    def _():
        part = jnp.sum((cnt_gt < _K).astype(jnp.float32)).reshape(1, 1)
        out_ref[...] += part

    @pl.when(ambiguous)
    def _():
        col = lax.broadcasted_iota(jnp.int32, (rb, n), 1)
        cnt_eq_low = jnp.sum(((pred == v) & (col < g)).astype(jnp.int32),
                             axis=1)
        part = jnp.sum(((cnt_gt + cnt_eq_low) < _K)
                       .astype(jnp.float32)).reshape(1, 1)
        out_ref[...] += part


def kernel(pred, gt):
    b, n = pred.shape
    rb = 64
    grid = (b // rb,)
    gt2 = gt.reshape(b, 1)
    out = pl.pallas_call(
        _acc_body,
        grid=grid,
        in_specs=[
            pl.BlockSpec((rb, 1), lambda i: (i, 0),
                         memory_space=pltpu.SMEM),
            pl.BlockSpec((rb, 1), lambda i: (i, 0)),
            pl.BlockSpec((rb, n), lambda i: (i, 0)),
        ],
        out_specs=pl.BlockSpec((1, 1), lambda i: (0, 0)),
        out_shape=jax.ShapeDtypeStruct((1, 1), jnp.float32),
    )(gt2, gt2, pred)
    return out[0, 0] / b
